# baseline (speedup 1.0000x reference)
import jax
import jax.numpy as jnp
from jax import lax
from jax.experimental import pallas as pl
from jax.experimental.pallas import tpu as pltpu
from jax.experimental.pallas import tpu_sc as plsc

N = 10000
D = 256
HD = 128
NP = 10240
NS = 16
RPT = NP // NS
E2 = 160000 + N
CH = 64
NCH = 176
ET = NCH * CH
EP = ET * NS
BN = 512


def _mm_body(x_ref, w_ref, va_ref, vd_ref, h0_ref, h1_ref, as_ref, ad_ref):
    h = jnp.dot(x_ref[...], w_ref[...], preferred_element_type=jnp.float32)
    h0_ref[...] = h[:, :HD]
    h1_ref[...] = h[:, HD:]
    as_ref[...] = jnp.dot(h, va_ref[...], preferred_element_type=jnp.float32)
    ad_ref[...] = jnp.dot(h, vd_ref[...], preferred_element_type=jnp.float32)


_mm = pl.pallas_call(
    _mm_body,
    grid=(NP // BN,),
    in_specs=[
        pl.BlockSpec((BN, D), lambda i: (i, 0)),
        pl.BlockSpec((D, D), lambda i: (0, 0)),
        pl.BlockSpec((D, 1), lambda i: (0, 0)),
        pl.BlockSpec((D, 1), lambda i: (0, 0)),
    ],
    out_specs=[
        pl.BlockSpec((BN, HD), lambda i: (i, 0)),
        pl.BlockSpec((BN, HD), lambda i: (i, 0)),
        pl.BlockSpec((BN, 1), lambda i: (i, 0)),
        pl.BlockSpec((BN, 1), lambda i: (i, 0)),
    ],
    out_shape=[
        jax.ShapeDtypeStruct((NP, HD), jnp.float32),
        jax.ShapeDtypeStruct((NP, HD), jnp.float32),
        jax.ShapeDtypeStruct((NP, 1), jnp.float32),
        jax.ShapeDtypeStruct((NP, 1), jnp.float32),
    ],
)


def _edge_body(src_ref, dst_ref, as_ref, ad_ref, hcat_ref,
               z2_ref, z1_ref, out0_ref, out1_ref, s_ref,
               si, di, sio, asg, adg, wv, rows,
               isem, agsem, gsem, ssem, osem, out_sh, s_sh):
    cid = lax.axis_index("c")
    sid = lax.axis_index("s")
    hoff = cid * NP
    ebase = sid * ET

    pltpu.sync_copy(z2_ref.at[pl.ds(sid * RPT, RPT)],
                    out_sh.at[pl.ds(sid * RPT, RPT)])

    @pl.when(sid == 0)
    def _():
        pltpu.sync_copy(z1_ref, s_sh)

    plsc.subcore_barrier()

    def start_idx(k, m8):
        base = ebase + k * CH
        pltpu.async_copy(src_ref.at[pl.ds(base, CH)], si.at[m8], isem.at[m8])
        pltpu.async_copy(dst_ref.at[pl.ds(base, CH)], di.at[m8], isem.at[m8])

    def wait_idx(m8):
        pltpu.make_async_copy(src_ref.at[pl.ds(0, CH)], si.at[m8],
                              isem.at[m8]).wait()
        pltpu.make_async_copy(dst_ref.at[pl.ds(0, CH)], di.at[m8],
                              isem.at[m8]).wait()

    def start_gathers(m8, b4):
        for j in range(CH // 16):
            sio[m8, pl.ds(j * 16, 16)] = si[m8, pl.ds(j * 16, 16)] + hoff
        pltpu.async_copy(as_ref.at[si.at[m8]], asg.at[b4], agsem.at[b4])
        pltpu.async_copy(ad_ref.at[di.at[m8]], adg.at[b4], agsem.at[b4])
        pltpu.async_copy(hcat_ref.at[sio.at[m8]], rows.at[b4], gsem.at[b4])

    def wait_gathers(m8, b4):
        pltpu.make_async_copy(as_ref.at[si.at[m8]], asg.at[b4],
                              agsem.at[b4]).wait()
        pltpu.make_async_copy(ad_ref.at[di.at[m8]], adg.at[b4],
                              agsem.at[b4]).wait()
        pltpu.make_async_copy(hcat_ref.at[sio.at[m8]], rows.at[b4],
                              gsem.at[b4]).wait()

    def wait_oscatter(b4, m8):
        pltpu.make_async_copy(rows.at[b4], out_sh.at[di.at[m8]],
                              osem.at[b4]).wait()

    def wait_sscatter(b4, m8):
        pltpu.make_async_copy(wv.at[b4], s_sh.at[di.at[m8]],
                              ssem.at[b4]).wait()

    def stage(k, m8, do_drain=True, do_idx=True, do_pref=True):
        b4 = m8 % 4
        c8 = (m8 + 2) % 8
        c4 = (m8 + 2) % 4
        n8 = (m8 + 4) % 8
        if do_drain:
            wait_oscatter(c4, c8)
            wait_sscatter(c4, c8)
        if do_idx:
            start_idx(k + 4, n8)
        if do_pref:
            wait_idx(c8)
            start_gathers(c8, c4)
        wait_gathers(m8, b4)
        for j in range(CH // 16):
            e = (asg[b4, pl.ds(j * 16, 16)] + adg[b4, pl.ds(j * 16, 16)])
            e = jnp.where(e > 0, e, 0.2 * e)
            wv[b4, pl.ds(j * 16, 16)] = jnp.exp(e)
        pltpu.async_copy(wv.at[b4], s_sh.at[di.at[m8]], ssem.at[b4], add=True)

        @plsc.parallel_loop(0, CH, 1, unroll=2)
        def _(r):
            wb = plsc.load_gather(wv, [jnp.zeros((16,), jnp.int32) + b4,
                                       jnp.zeros((16,), jnp.int32) + r])
            for f in range(HD // 16):
                rows[b4, r, pl.ds(f * 16, 16)] = (
                    rows[b4, r, pl.ds(f * 16, 16)] * wb)

        pltpu.async_copy(rows.at[b4], out_sh.at[di.at[m8]], osem.at[b4],
                         add=True)

    for t in range(4):
        start_idx(t, t)
    for t in range(2):
        wait_idx(t)
        start_gathers(t, t)
    stage(0, 0, do_drain=False)
    stage(1, 1, do_drain=False)

    def body(jj, _):
        k0 = 2 + jj * 8
        for t in range(8):
            stage(k0 + t, (2 + t) % 8)
        return 0

    lax.fori_loop(0, (NCH - 8) // 8, body, 0)
    stage(NCH - 6, (NCH - 6) % 8)
    stage(NCH - 5, (NCH - 5) % 8)
    stage(NCH - 4, (NCH - 4) % 8, do_idx=False)
    stage(NCH - 3, (NCH - 3) % 8, do_idx=False)
    stage(NCH - 2, (NCH - 2) % 8, do_idx=False, do_pref=False)
    stage(NCH - 1, (NCH - 1) % 8, do_idx=False, do_pref=False)
    wait_oscatter((NCH - 2) % 4, (NCH - 2) % 8)
    wait_sscatter((NCH - 2) % 4, (NCH - 2) % 8)
    wait_oscatter((NCH - 1) % 4, (NCH - 1) % 8)
    wait_sscatter((NCH - 1) % 4, (NCH - 1) % 8)
    plsc.subcore_barrier()

    @pl.when(cid == 0)
    def _():
        pltpu.sync_copy(out_sh.at[pl.ds(sid * RPT, RPT)],
                        out0_ref.at[pl.ds(sid * RPT, RPT)])

    @pl.when(cid == 1)
    def _():
        pltpu.sync_copy(out_sh.at[pl.ds(sid * RPT, RPT)],
                        out1_ref.at[pl.ds(sid * RPT, RPT)])

    @pl.when(jnp.logical_and(cid == 0, sid == 0))
    def _():
        pltpu.sync_copy(s_sh, s_ref)


_edge = pl.kernel(
    _edge_body,
    out_type=[
        jax.ShapeDtypeStruct((NP, HD), jnp.float32),
        jax.ShapeDtypeStruct((NP, HD), jnp.float32),
        jax.ShapeDtypeStruct((NP,), jnp.float32),
    ],
    mesh=plsc.VectorSubcoreMesh(core_axis_name="c", subcore_axis_name="s"),
    compiler_params=pltpu.CompilerParams(needs_layout_passes=False),
    scratch_types=[
        pltpu.VMEM((8, CH), jnp.int32),
        pltpu.VMEM((8, CH), jnp.int32),
        pltpu.VMEM((8, CH), jnp.int32),
        pltpu.VMEM((4, CH), jnp.float32),
        pltpu.VMEM((4, CH), jnp.float32),
        pltpu.VMEM((4, CH), jnp.float32),
        pltpu.VMEM((4, CH, HD), jnp.float32),
        pltpu.SemaphoreType.DMA((8,)),
        pltpu.SemaphoreType.DMA((4,)),
        pltpu.SemaphoreType.DMA((4,)),
        pltpu.SemaphoreType.DMA((4,)),
        pltpu.SemaphoreType.DMA((4,)),
        pltpu.VMEM_SHARED((NP, HD), jnp.float32),
        pltpu.VMEM_SHARED((NP,), jnp.float32),
    ],
)


def _ln_body(x_ref, o0_ref, o1_ref, s_ref, b_ref, g_ref, be_ref, y_ref):
    inv = 1.0 / (s_ref[...] + 1e-16)
    att = jnp.concatenate([o0_ref[...] * inv, o1_ref[...] * inv], axis=1)
    t = x_ref[...] + att + b_ref[...]
    mu = jnp.mean(t, axis=1, keepdims=True)
    var = jnp.mean((t - mu) ** 2, axis=1, keepdims=True)
    y_ref[...] = (t - mu) * lax.rsqrt(var + 1e-5) * g_ref[...] + be_ref[...]


_ln = pl.pallas_call(
    _ln_body,
    grid=(NP // BN,),
    in_specs=[
        pl.BlockSpec((BN, D), lambda i: (i, 0)),
        pl.BlockSpec((BN, HD), lambda i: (i, 0)),
        pl.BlockSpec((BN, HD), lambda i: (i, 0)),
        pl.BlockSpec((BN, 1), lambda i: (i, 0)),
        pl.BlockSpec((1, D), lambda i: (0, 0)),
        pl.BlockSpec((1, D), lambda i: (0, 0)),
        pl.BlockSpec((1, D), lambda i: (0, 0)),
    ],
    out_specs=pl.BlockSpec((BN, D), lambda i: (i, 0)),
    out_shape=jax.ShapeDtypeStruct((NP, D), jnp.float32),
)


@jax.jit
def _run(x, edge_index, W0, a_src0, a_dst0, b0, g0, be0,
         W1, a_src1, a_dst1, b1, g1, be1):
    xp = jnp.zeros((NP, D), jnp.float32).at[:N].set(x)
    ar = jnp.arange(N, dtype=jnp.int32)
    pad = jnp.full((EP - E2,), NP - 1, jnp.int32)
    src = jnp.concatenate([edge_index[0].astype(jnp.int32), ar, pad])
    dst = jnp.concatenate([edge_index[1].astype(jnp.int32), ar, pad])
    z2 = jnp.zeros((NP, HD), jnp.float32)
    z1 = jnp.zeros((NP,), jnp.float32)

    for (W, a_s, a_d, b, g, be) in (
            (W0, a_src0, a_dst0, b0, g0, be0),
            (W1, a_src1, a_dst1, b1, g1, be1)):
        va = a_s.reshape(D, 1)
        vd = a_d.reshape(D, 1)
        h0, h1, asrc, adst = _mm(xp, W, va, vd)
        hcat = jnp.concatenate([h0, h1], axis=0)
        out0, out1, s = _edge(src, dst, asrc.reshape(NP), adst.reshape(NP),
                              hcat, z2, z1)
        xp = _ln(xp, out0, out1, s.reshape(NP, 1),
                 b.reshape(1, D), g.reshape(1, D), be.reshape(1, D))
    return xp[:N]


def kernel(x, edge_index, W0, a_src0, a_dst0, b0, g0, be0,
           W1, a_src1, a_dst1, b1, g1, be1):
    return _run(x, edge_index, W0, a_src0, a_dst0, b0, g0, be0,
                W1, a_src1, a_dst1, b1, g1, be1)


# speedup vs baseline: 1.4537x; 1.4537x over previous
import jax
import jax.numpy as jnp
from jax import lax
from jax.experimental import pallas as pl
from jax.experimental.pallas import tpu as pltpu
from jax.experimental.pallas import tpu_sc as plsc

N = 10000
D = 256
HD = 128
NP = 10240
NS = 16
RPT = NP // NS
E2 = 160000 + N
CH = 128
ET = ((E2 + NS * CH - 1) // (NS * CH)) * CH
EP = ET * NS
NCH = ET // CH
BN = 512


def _mm_body(x_ref, w_ref, va_ref, vd_ref, h0_ref, h1_ref, as_ref, ad_ref):
    h = jnp.dot(x_ref[...], w_ref[...], preferred_element_type=jnp.float32)
    h0_ref[...] = h[:, :HD]
    h1_ref[...] = h[:, HD:]
    as_ref[...] = jnp.dot(h, va_ref[...], preferred_element_type=jnp.float32)
    ad_ref[...] = jnp.dot(h, vd_ref[...], preferred_element_type=jnp.float32)


_mm = pl.pallas_call(
    _mm_body,
    grid=(NP // BN,),
    in_specs=[
        pl.BlockSpec((BN, D), lambda i: (i, 0)),
        pl.BlockSpec((D, D), lambda i: (0, 0)),
        pl.BlockSpec((D, 1), lambda i: (0, 0)),
        pl.BlockSpec((D, 1), lambda i: (0, 0)),
    ],
    out_specs=[
        pl.BlockSpec((BN, HD), lambda i: (i, 0)),
        pl.BlockSpec((BN, HD), lambda i: (i, 0)),
        pl.BlockSpec((BN, 1), lambda i: (i, 0)),
        pl.BlockSpec((BN, 1), lambda i: (i, 0)),
    ],
    out_shape=[
        jax.ShapeDtypeStruct((NP, HD), jnp.float32),
        jax.ShapeDtypeStruct((NP, HD), jnp.float32),
        jax.ShapeDtypeStruct((NP, 1), jnp.float32),
        jax.ShapeDtypeStruct((NP, 1), jnp.float32),
    ],
)


def _edge_body(src_ref, dst_ref, as_ref, ad_ref, h0_ref, h1_ref,
               z2_ref, z1_ref, out0_ref, out1_ref, s_ref,
               av, dvv, si, di, wv, rows, sem, out_sh, s_sh):
    cid = lax.axis_index("c")
    sid = lax.axis_index("s")

    pltpu.sync_copy(z2_ref.at[pl.ds(sid * RPT, RPT)],
                    out_sh.at[pl.ds(sid * RPT, RPT)])

    @pl.when(sid == 0)
    def _():
        pltpu.sync_copy(z1_ref, s_sh)

    pltpu.sync_copy(as_ref, av)
    pltpu.sync_copy(ad_ref, dvv)
    plsc.subcore_barrier()

    def run_half(h_ref):
        def chunk_body(k, _):
            base = sid * ET + k * CH
            pltpu.sync_copy(src_ref.at[pl.ds(base, CH)], si)
            pltpu.sync_copy(dst_ref.at[pl.ds(base, CH)], di)
            for j in range(CH // 16):
                sv = si[pl.ds(j * 16, 16)]
                dv = di[pl.ds(j * 16, 16)]
                e = plsc.load_gather(av, [sv]) + plsc.load_gather(dvv, [dv])
                e = jnp.where(e > 0, e, 0.2 * e)
                wv[pl.ds(j * 16, 16)] = jnp.exp(e)
            pltpu.sync_copy(wv, s_sh.at[di], add=True)
            pltpu.async_copy(h_ref.at[si], rows, sem).wait()

            @plsc.parallel_loop(0, CH, 1, unroll=4)
            def _(r):
                wb = plsc.load_gather(wv, [jnp.zeros((16,), jnp.int32) + r])
                for f in range(HD // 16):
                    rows[r, pl.ds(f * 16, 16)] = (
                        rows[r, pl.ds(f * 16, 16)] * wb)

            pltpu.sync_copy(rows, out_sh.at[di], add=True)
            return 0

        lax.fori_loop(0, NCH, chunk_body, 0)

    @pl.when(cid == 0)
    def _():
        run_half(h0_ref)

    @pl.when(cid == 1)
    def _():
        run_half(h1_ref)

    plsc.subcore_barrier()

    @pl.when(cid == 0)
    def _():
        pltpu.sync_copy(out_sh.at[pl.ds(sid * RPT, RPT)],
                        out0_ref.at[pl.ds(sid * RPT, RPT)])

    @pl.when(cid == 1)
    def _():
        pltpu.sync_copy(out_sh.at[pl.ds(sid * RPT, RPT)],
                        out1_ref.at[pl.ds(sid * RPT, RPT)])

    @pl.when(jnp.logical_and(cid == 0, sid == 0))
    def _():
        pltpu.sync_copy(s_sh, s_ref)


_edge = pl.kernel(
    _edge_body,
    out_type=[
        jax.ShapeDtypeStruct((NP, HD), jnp.float32),
        jax.ShapeDtypeStruct((NP, HD), jnp.float32),
        jax.ShapeDtypeStruct((NP,), jnp.float32),
    ],
    mesh=plsc.VectorSubcoreMesh(core_axis_name="c", subcore_axis_name="s"),
    compiler_params=pltpu.CompilerParams(needs_layout_passes=False),
    scratch_types=[
        pltpu.VMEM((NP,), jnp.float32),
        pltpu.VMEM((NP,), jnp.float32),
        pltpu.VMEM((CH,), jnp.int32),
        pltpu.VMEM((CH,), jnp.int32),
        pltpu.VMEM((CH,), jnp.float32),
        pltpu.VMEM((CH, HD), jnp.float32),
        pltpu.SemaphoreType.DMA,
        pltpu.VMEM_SHARED((NP, HD), jnp.float32),
        pltpu.VMEM_SHARED((NP,), jnp.float32),
    ],
)


def _ln_body(x_ref, o0_ref, o1_ref, s_ref, b_ref, g_ref, be_ref, y_ref):
    inv = 1.0 / (s_ref[...] + 1e-16)
    att = jnp.concatenate([o0_ref[...] * inv, o1_ref[...] * inv], axis=1)
    t = x_ref[...] + att + b_ref[...]
    mu = jnp.mean(t, axis=1, keepdims=True)
    var = jnp.mean((t - mu) ** 2, axis=1, keepdims=True)
    y_ref[...] = (t - mu) * lax.rsqrt(var + 1e-5) * g_ref[...] + be_ref[...]


_ln = pl.pallas_call(
    _ln_body,
    grid=(NP // BN,),
    in_specs=[
        pl.BlockSpec((BN, D), lambda i: (i, 0)),
        pl.BlockSpec((BN, HD), lambda i: (i, 0)),
        pl.BlockSpec((BN, HD), lambda i: (i, 0)),
        pl.BlockSpec((BN, 1), lambda i: (i, 0)),
        pl.BlockSpec((1, D), lambda i: (0, 0)),
        pl.BlockSpec((1, D), lambda i: (0, 0)),
        pl.BlockSpec((1, D), lambda i: (0, 0)),
    ],
    out_specs=pl.BlockSpec((BN, D), lambda i: (i, 0)),
    out_shape=jax.ShapeDtypeStruct((NP, D), jnp.float32),
)


@jax.jit
def _run(x, edge_index, W0, a_src0, a_dst0, b0, g0, be0,
         W1, a_src1, a_dst1, b1, g1, be1):
    xp = jnp.zeros((NP, D), jnp.float32).at[:N].set(x)
    ar = jnp.arange(N, dtype=jnp.int32)
    pad = jnp.full((EP - E2,), NP - 1, jnp.int32)
    src = jnp.concatenate([edge_index[0].astype(jnp.int32), ar, pad])
    dst = jnp.concatenate([edge_index[1].astype(jnp.int32), ar, pad])
    z2 = jnp.zeros((NP, HD), jnp.float32)
    z1 = jnp.zeros((NP,), jnp.float32)

    for (W, a_s, a_d, b, g, be) in (
            (W0, a_src0, a_dst0, b0, g0, be0),
            (W1, a_src1, a_dst1, b1, g1, be1)):
        va = a_s.reshape(D, 1)
        vd = a_d.reshape(D, 1)
        h0, h1, asrc, adst = _mm(xp, W, va, vd)
        out0, out1, s = _edge(src, dst, asrc.reshape(NP), adst.reshape(NP),
                              h0, h1, z2, z1)
        xp = _ln(xp, out0, out1, s.reshape(NP, 1),
                 b.reshape(1, D), g.reshape(1, D), be.reshape(1, D))
    return xp[:N]


def kernel(x, edge_index, W0, a_src0, a_dst0, b0, g0, be0,
           W1, a_src1, a_dst1, b1, g1, be1):
    return _run(x, edge_index, W0, a_src0, a_dst0, b0, g0, be0,
                W1, a_src1, a_dst1, b1, g1, be1)


# speedup vs baseline: 1.6416x; 1.1293x over previous
import jax
import jax.numpy as jnp
from jax import lax
from jax.experimental import pallas as pl
from jax.experimental.pallas import tpu as pltpu
from jax.experimental.pallas import tpu_sc as plsc

N = 10000
D = 256
HD = 128
NP = 10112
NS = 16
RPT = NP // NS
E2 = 160000 + N
CH = 80
NCH = 137
ET = NCH * CH
EP = ET * NS
BN = 632


def _mm_body(x_ref, w_ref, va_ref, vd_ref, h0_ref, h1_ref, as_ref, ad_ref):
    h = jnp.dot(x_ref[...], w_ref[...], preferred_element_type=jnp.float32)
    h0_ref[...] = h[:, :HD]
    h1_ref[...] = h[:, HD:]
    as_ref[...] = jnp.dot(h, va_ref[...], preferred_element_type=jnp.float32)
    ad_ref[...] = jnp.dot(h, vd_ref[...], preferred_element_type=jnp.float32)


_mm = pl.pallas_call(
    _mm_body,
    grid=(NP // BN,),
    in_specs=[
        pl.BlockSpec((BN, D), lambda i: (i, 0)),
        pl.BlockSpec((D, D), lambda i: (0, 0)),
        pl.BlockSpec((D, 1), lambda i: (0, 0)),
        pl.BlockSpec((D, 1), lambda i: (0, 0)),
    ],
    out_specs=[
        pl.BlockSpec((BN, HD), lambda i: (i, 0)),
        pl.BlockSpec((BN, HD), lambda i: (i, 0)),
        pl.BlockSpec((BN, 1), lambda i: (i, 0)),
        pl.BlockSpec((BN, 1), lambda i: (i, 0)),
    ],
    out_shape=[
        jax.ShapeDtypeStruct((NP, HD), jnp.float32),
        jax.ShapeDtypeStruct((NP, HD), jnp.float32),
        jax.ShapeDtypeStruct((NP, 1), jnp.float32),
        jax.ShapeDtypeStruct((NP, 1), jnp.float32),
    ],
)


def _edge_body(src_ref, dst_ref, as_ref, ad_ref, hcat_ref,
               z2_ref, z1_ref, out0_ref, out1_ref, s_ref,
               av, si, di, sio, adg, wv, rows,
               isem, agsem, gsem, ssem, osem, out_sh, s_sh):
    cid = lax.axis_index("c")
    sid = lax.axis_index("s")
    hoff = cid * NP
    ebase = sid * ET

    pltpu.sync_copy(z2_ref.at[pl.ds(sid * RPT, RPT)],
                    out_sh.at[pl.ds(sid * RPT, RPT)])

    @pl.when(sid == 0)
    def _():
        pltpu.sync_copy(z1_ref, s_sh)

    pltpu.sync_copy(as_ref, av)
    plsc.subcore_barrier()

    def start_idx(k, m6):
        base = ebase + k * CH
        pltpu.async_copy(src_ref.at[pl.ds(base, CH)], si.at[m6], isem.at[m6])
        pltpu.async_copy(dst_ref.at[pl.ds(base, CH)], di.at[m6], isem.at[m6])

    def wait_idx(m6):
        pltpu.make_async_copy(src_ref.at[pl.ds(0, CH)], si.at[m6],
                              isem.at[m6]).wait()
        pltpu.make_async_copy(dst_ref.at[pl.ds(0, CH)], di.at[m6],
                              isem.at[m6]).wait()

    def start_gathers(m6, b3):
        for j in range(CH // 16):
            sio[m6, pl.ds(j * 16, 16)] = si[m6, pl.ds(j * 16, 16)] + hoff
        pltpu.async_copy(ad_ref.at[di.at[m6]], adg.at[b3], agsem.at[b3])
        pltpu.async_copy(hcat_ref.at[sio.at[m6]], rows.at[b3], gsem.at[b3])

    def wait_gathers(m6, b3):
        pltpu.make_async_copy(ad_ref.at[di.at[m6]], adg.at[b3],
                              agsem.at[b3]).wait()
        pltpu.make_async_copy(hcat_ref.at[sio.at[m6]], rows.at[b3],
                              gsem.at[b3]).wait()

    def wait_oscatter(b3, m6):
        pltpu.make_async_copy(rows.at[b3], out_sh.at[di.at[m6]],
                              osem.at[b3]).wait()

    def wait_sscatter(b3, m6):
        pltpu.make_async_copy(wv.at[b3], s_sh.at[di.at[m6]],
                              ssem.at[b3]).wait()

    def stage(k, m6, do_drain=True, do_idx=True, do_pref=True):
        b3 = m6 % 3
        n6 = (m6 + 1) % 6
        n3 = (m6 + 1) % 3
        p6 = (m6 + 2) % 6
        if do_drain:
            wait_oscatter(n3, n6)
            wait_sscatter(n3, n6)
        if do_idx:
            start_idx(k + 2, p6)
        if do_pref:
            wait_idx(n6)
            start_gathers(n6, n3)
        wait_gathers(m6, b3)
        for j in range(CH // 16):
            sv = si[m6, pl.ds(j * 16, 16)]
            e = plsc.load_gather(av, [sv]) + adg[b3, pl.ds(j * 16, 16)]
            e = jnp.where(e > 0, e, 0.2 * e)
            wv[b3, pl.ds(j * 16, 16)] = jnp.exp(e)
        pltpu.async_copy(wv.at[b3], s_sh.at[di.at[m6]], ssem.at[b3], add=True)

        @plsc.parallel_loop(0, CH, 1, unroll=4)
        def _(r):
            wb = plsc.load_gather(wv, [jnp.zeros((16,), jnp.int32) + b3,
                                       jnp.zeros((16,), jnp.int32) + r])
            for f in range(HD // 16):
                rows[b3, r, pl.ds(f * 16, 16)] = (
                    rows[b3, r, pl.ds(f * 16, 16)] * wb)

        pltpu.async_copy(rows.at[b3], out_sh.at[di.at[m6]], osem.at[b3],
                         add=True)

    start_idx(0, 0)
    start_idx(1, 1)
    wait_idx(0)
    start_gathers(0, 0)
    stage(0, 0, do_drain=False)
    stage(1, 1, do_drain=False)

    def body(jj, _):
        k0 = 2 + jj * 6
        for t in range(6):
            stage(k0 + t, (2 + t) % 6)
        return 0

    lax.fori_loop(0, (NCH - 5) // 6, body, 0)
    stage(NCH - 3, (NCH - 3) % 6)
    stage(NCH - 2, (NCH - 2) % 6, do_idx=False)
    stage(NCH - 1, (NCH - 1) % 6, do_idx=False, do_pref=False)
    wait_oscatter((NCH - 2) % 3, (NCH - 2) % 6)
    wait_sscatter((NCH - 2) % 3, (NCH - 2) % 6)
    wait_oscatter((NCH - 1) % 3, (NCH - 1) % 6)
    wait_sscatter((NCH - 1) % 3, (NCH - 1) % 6)
    plsc.subcore_barrier()

    @pl.when(cid == 0)
    def _():
        pltpu.sync_copy(out_sh.at[pl.ds(sid * RPT, RPT)],
                        out0_ref.at[pl.ds(sid * RPT, RPT)])

    @pl.when(cid == 1)
    def _():
        pltpu.sync_copy(out_sh.at[pl.ds(sid * RPT, RPT)],
                        out1_ref.at[pl.ds(sid * RPT, RPT)])

    @pl.when(jnp.logical_and(cid == 0, sid == 0))
    def _():
        pltpu.sync_copy(s_sh, s_ref)


_edge = pl.kernel(
    _edge_body,
    out_type=[
        jax.ShapeDtypeStruct((NP, HD), jnp.float32),
        jax.ShapeDtypeStruct((NP, HD), jnp.float32),
        jax.ShapeDtypeStruct((NP,), jnp.float32),
    ],
    mesh=plsc.VectorSubcoreMesh(core_axis_name="c", subcore_axis_name="s"),
    compiler_params=pltpu.CompilerParams(needs_layout_passes=False),
    scratch_types=[
        pltpu.VMEM((NP,), jnp.float32),
        pltpu.VMEM((6, CH), jnp.int32),
        pltpu.VMEM((6, CH), jnp.int32),
        pltpu.VMEM((6, CH), jnp.int32),
        pltpu.VMEM((3, CH), jnp.float32),
        pltpu.VMEM((3, CH), jnp.float32),
        pltpu.VMEM((3, CH, HD), jnp.float32),
        pltpu.SemaphoreType.DMA((6,)),
        pltpu.SemaphoreType.DMA((3,)),
        pltpu.SemaphoreType.DMA((3,)),
        pltpu.SemaphoreType.DMA((3,)),
        pltpu.SemaphoreType.DMA((3,)),
        pltpu.VMEM_SHARED((NP, HD), jnp.float32),
        pltpu.VMEM_SHARED((NP,), jnp.float32),
    ],
)


def _ln_body(x_ref, o0_ref, o1_ref, s_ref, b_ref, g_ref, be_ref, y_ref):
    inv = 1.0 / (s_ref[...] + 1e-16)
    att = jnp.concatenate([o0_ref[...] * inv, o1_ref[...] * inv], axis=1)
    t = x_ref[...] + att + b_ref[...]
    mu = jnp.mean(t, axis=1, keepdims=True)
    var = jnp.mean((t - mu) ** 2, axis=1, keepdims=True)
    y_ref[...] = (t - mu) * lax.rsqrt(var + 1e-5) * g_ref[...] + be_ref[...]


_ln = pl.pallas_call(
    _ln_body,
    grid=(NP // BN,),
    in_specs=[
        pl.BlockSpec((BN, D), lambda i: (i, 0)),
        pl.BlockSpec((BN, HD), lambda i: (i, 0)),
        pl.BlockSpec((BN, HD), lambda i: (i, 0)),
        pl.BlockSpec((BN, 1), lambda i: (i, 0)),
        pl.BlockSpec((1, D), lambda i: (0, 0)),
        pl.BlockSpec((1, D), lambda i: (0, 0)),
        pl.BlockSpec((1, D), lambda i: (0, 0)),
    ],
    out_specs=pl.BlockSpec((BN, D), lambda i: (i, 0)),
    out_shape=jax.ShapeDtypeStruct((NP, D), jnp.float32),
)


@jax.jit
def _run(x, edge_index, W0, a_src0, a_dst0, b0, g0, be0,
         W1, a_src1, a_dst1, b1, g1, be1):
    xp = jnp.zeros((NP, D), jnp.float32).at[:N].set(x)
    ar = jnp.arange(N, dtype=jnp.int32)
    pad = jnp.full((EP - E2,), NP - 1, jnp.int32)
    src = jnp.concatenate([edge_index[0].astype(jnp.int32), ar, pad])
    dst = jnp.concatenate([edge_index[1].astype(jnp.int32), ar, pad])
    z2 = jnp.zeros((NP, HD), jnp.float32)
    z1 = jnp.zeros((NP,), jnp.float32)

    for (W, a_s, a_d, b, g, be) in (
            (W0, a_src0, a_dst0, b0, g0, be0),
            (W1, a_src1, a_dst1, b1, g1, be1)):
        va = a_s.reshape(D, 1)
        vd = a_d.reshape(D, 1)
        h0, h1, asrc, adst = _mm(xp, W, va, vd)
        hcat = jnp.concatenate([h0, h1], axis=0)
        out0, out1, s = _edge(src, dst, asrc.reshape(NP), adst.reshape(NP),
                              hcat, z2, z1)
        xp = _ln(xp, out0, out1, s.reshape(NP, 1),
                 b.reshape(1, D), g.reshape(1, D), be.reshape(1, D))
    return xp[:N]


def kernel(x, edge_index, W0, a_src0, a_dst0, b0, g0, be0,
           W1, a_src1, a_dst1, b1, g1, be1):
    return _run(x, edge_index, W0, a_src0, a_dst0, b0, g0, be0,
                W1, a_src1, a_dst1, b1, g1, be1)


# speedup vs baseline: 3.3704x; 2.0531x over previous
import jax
import jax.numpy as jnp
from jax import lax
from jax.experimental import pallas as pl
from jax.experimental.pallas import tpu as pltpu
from jax.experimental.pallas import tpu_sc as plsc

N = 10000
D = 256
HD = 128
NP = 10112
NS = 16
RPT = NP // NS
E = 160000
CH = 80
NCH = 125
ET = NCH * CH
BN = 632


def _mm_body(x_ref, w_ref, va_ref, vd_ref, h0_ref, h1_ref, as_ref, ad_ref):
    h = jnp.dot(x_ref[...], w_ref[...], preferred_element_type=jnp.float32)
    h0_ref[...] = h[:, :HD]
    h1_ref[...] = h[:, HD:]
    as_ref[...] = jnp.dot(h, va_ref[...], preferred_element_type=jnp.float32)
    ad_ref[...] = jnp.dot(h, vd_ref[...], preferred_element_type=jnp.float32)


_mm = pl.pallas_call(
    _mm_body,
    grid=(NP // BN,),
    in_specs=[
        pl.BlockSpec((BN, D), lambda i: (i, 0)),
        pl.BlockSpec((D, D), lambda i: (0, 0)),
        pl.BlockSpec((D, 1), lambda i: (0, 0)),
        pl.BlockSpec((D, 1), lambda i: (0, 0)),
    ],
    out_specs=[
        pl.BlockSpec((BN, HD), lambda i: (i, 0)),
        pl.BlockSpec((BN, HD), lambda i: (i, 0)),
        pl.BlockSpec((BN, 1), lambda i: (i, 0)),
        pl.BlockSpec((BN, 1), lambda i: (i, 0)),
    ],
    out_shape=[
        jax.ShapeDtypeStruct((NP, HD), jnp.float32),
        jax.ShapeDtypeStruct((NP, HD), jnp.float32),
        jax.ShapeDtypeStruct((NP, 1), jnp.float32),
        jax.ShapeDtypeStruct((NP, 1), jnp.float32),
    ],
)


def _edge_body(src_ref, dst_ref, as_ref, ad_ref, hcat_ref,
               z2_ref, z1_ref, out0_ref, out1_ref, s_ref,
               av, si, di, sio, adg, wv, rows,
               isem, agsem, gsem, ssem, osem, out_sh, s_sh):
    cid = lax.axis_index("c")
    sid = lax.axis_index("s")
    hoff = cid * NP
    ebase = sid * ET

    pltpu.sync_copy(z2_ref.at[pl.ds(sid * RPT, RPT)],
                    out_sh.at[pl.ds(sid * RPT, RPT)])

    @pl.when(sid == 0)
    def _():
        pltpu.sync_copy(z1_ref, s_sh)

    pltpu.sync_copy(as_ref, av)
    plsc.subcore_barrier()

    def start_idx(k, m6):
        base = ebase + k * CH
        pltpu.async_copy(src_ref.at[pl.ds(base, CH)], si.at[m6], isem.at[m6])
        pltpu.async_copy(dst_ref.at[pl.ds(base, CH)], di.at[m6], isem.at[m6])

    def wait_idx(m6):
        pltpu.make_async_copy(src_ref.at[pl.ds(0, CH)], si.at[m6],
                              isem.at[m6]).wait()
        pltpu.make_async_copy(dst_ref.at[pl.ds(0, CH)], di.at[m6],
                              isem.at[m6]).wait()

    def start_gathers(m6, b3):
        for j in range(CH // 16):
            sio[m6, pl.ds(j * 16, 16)] = si[m6, pl.ds(j * 16, 16)] + hoff
        pltpu.async_copy(ad_ref.at[di.at[m6]], adg.at[b3], agsem.at[b3])
        pltpu.async_copy(hcat_ref.at[sio.at[m6]], rows.at[b3], gsem.at[b3])

    def wait_gathers(m6, b3):
        pltpu.make_async_copy(ad_ref.at[di.at[m6]], adg.at[b3],
                              agsem.at[b3]).wait()
        pltpu.make_async_copy(hcat_ref.at[sio.at[m6]], rows.at[b3],
                              gsem.at[b3]).wait()

    def wait_oscatter(b3, m6):
        pltpu.make_async_copy(rows.at[b3], out_sh.at[di.at[m6]],
                              osem.at[b3]).wait()

    def wait_sscatter(b3, m6):
        pltpu.make_async_copy(wv.at[b3], s_sh.at[di.at[m6]],
                              ssem.at[b3]).wait()

    def stage(k, m6, do_drain=True, do_idx=True, do_pref=True):
        b3 = m6 % 3
        n6 = (m6 + 1) % 6
        n3 = (m6 + 1) % 3
        p6 = (m6 + 2) % 6
        if do_drain:
            wait_oscatter(n3, n6)
            wait_sscatter(n3, n6)
        if do_idx:
            start_idx(k + 2, p6)
        if do_pref:
            wait_idx(n6)
            start_gathers(n6, n3)
        wait_gathers(m6, b3)
        for j in range(CH // 16):
            sv = si[m6, pl.ds(j * 16, 16)]
            e = plsc.load_gather(av, [sv]) + adg[b3, pl.ds(j * 16, 16)]
            e = jnp.where(e > 0, e, 0.2 * e)
            wv[b3, pl.ds(j * 16, 16)] = jnp.exp(e)
        pltpu.async_copy(wv.at[b3], s_sh.at[di.at[m6]], ssem.at[b3], add=True)

        @plsc.parallel_loop(0, CH, 1, unroll=4)
        def _(r):
            wb = plsc.load_gather(wv, [jnp.zeros((16,), jnp.int32) + b3,
                                       jnp.zeros((16,), jnp.int32) + r])
            for f in range(HD // 16):
                rows[b3, r, pl.ds(f * 16, 16)] = (
                    rows[b3, r, pl.ds(f * 16, 16)] * wb)

        pltpu.async_copy(rows.at[b3], out_sh.at[di.at[m6]], osem.at[b3],
                         add=True)

    start_idx(0, 0)
    start_idx(1, 1)
    wait_idx(0)
    start_gathers(0, 0)
    stage(0, 0, do_drain=False)
    stage(1, 1, do_drain=False)

    def body(jj, _):
        k0 = 2 + jj * 6
        for t in range(6):
            stage(k0 + t, (2 + t) % 6)
        return 0

    lax.fori_loop(0, (NCH - 5) // 6, body, 0)
    stage(NCH - 3, (NCH - 3) % 6)
    stage(NCH - 2, (NCH - 2) % 6, do_idx=False)
    stage(NCH - 1, (NCH - 1) % 6, do_idx=False, do_pref=False)
    wait_oscatter((NCH - 2) % 3, (NCH - 2) % 6)
    wait_sscatter((NCH - 2) % 3, (NCH - 2) % 6)
    wait_oscatter((NCH - 1) % 3, (NCH - 1) % 6)
    wait_sscatter((NCH - 1) % 3, (NCH - 1) % 6)
    plsc.subcore_barrier()

    @pl.when(cid == 0)
    def _():
        pltpu.sync_copy(out_sh.at[pl.ds(sid * RPT, RPT)],
                        out0_ref.at[pl.ds(sid * RPT, RPT)])

    @pl.when(cid == 1)
    def _():
        pltpu.sync_copy(out_sh.at[pl.ds(sid * RPT, RPT)],
                        out1_ref.at[pl.ds(sid * RPT, RPT)])

    @pl.when(jnp.logical_and(cid == 0, sid == 0))
    def _():
        pltpu.sync_copy(s_sh, s_ref)


_edge = pl.kernel(
    _edge_body,
    out_type=[
        jax.ShapeDtypeStruct((NP, HD), jnp.float32),
        jax.ShapeDtypeStruct((NP, HD), jnp.float32),
        jax.ShapeDtypeStruct((NP,), jnp.float32),
    ],
    mesh=plsc.VectorSubcoreMesh(core_axis_name="c", subcore_axis_name="s"),
    compiler_params=pltpu.CompilerParams(needs_layout_passes=False),
    scratch_types=[
        pltpu.VMEM((NP,), jnp.float32),
        pltpu.VMEM((6, CH), jnp.int32),
        pltpu.VMEM((6, CH), jnp.int32),
        pltpu.VMEM((6, CH), jnp.int32),
        pltpu.VMEM((3, CH), jnp.float32),
        pltpu.VMEM((3, CH), jnp.float32),
        pltpu.VMEM((3, CH, HD), jnp.float32),
        pltpu.SemaphoreType.DMA((6,)),
        pltpu.SemaphoreType.DMA((3,)),
        pltpu.SemaphoreType.DMA((3,)),
        pltpu.SemaphoreType.DMA((3,)),
        pltpu.SemaphoreType.DMA((3,)),
        pltpu.VMEM_SHARED((NP, HD), jnp.float32),
        pltpu.VMEM_SHARED((NP,), jnp.float32),
    ],
)


def _ln_body(x_ref, o0_ref, o1_ref, h0_ref, h1_ref, s_ref, as_ref, ad_ref,
             b_ref, g_ref, be_ref, y_ref):
    es = as_ref[...] + ad_ref[...]
    ws = jnp.exp(jnp.where(es > 0, es, 0.2 * es))
    inv = 1.0 / (s_ref[...] + ws + 1e-16)
    att = jnp.concatenate(
        [(o0_ref[...] + ws * h0_ref[...]) * inv,
         (o1_ref[...] + ws * h1_ref[...]) * inv], axis=1)
    t = x_ref[...] + att + b_ref[...]
    mu = jnp.mean(t, axis=1, keepdims=True)
    var = jnp.mean((t - mu) ** 2, axis=1, keepdims=True)
    y_ref[...] = (t - mu) * lax.rsqrt(var + 1e-5) * g_ref[...] + be_ref[...]


_ln = pl.pallas_call(
    _ln_body,
    grid=(NP // BN,),
    in_specs=[
        pl.BlockSpec((BN, D), lambda i: (i, 0)),
        pl.BlockSpec((BN, HD), lambda i: (i, 0)),
        pl.BlockSpec((BN, HD), lambda i: (i, 0)),
        pl.BlockSpec((BN, HD), lambda i: (i, 0)),
        pl.BlockSpec((BN, HD), lambda i: (i, 0)),
        pl.BlockSpec((BN, 1), lambda i: (i, 0)),
        pl.BlockSpec((BN, 1), lambda i: (i, 0)),
        pl.BlockSpec((BN, 1), lambda i: (i, 0)),
        pl.BlockSpec((1, D), lambda i: (0, 0)),
        pl.BlockSpec((1, D), lambda i: (0, 0)),
        pl.BlockSpec((1, D), lambda i: (0, 0)),
    ],
    out_specs=pl.BlockSpec((BN, D), lambda i: (i, 0)),
    out_shape=jax.ShapeDtypeStruct((NP, D), jnp.float32),
)


@jax.jit
def _run(x, edge_index, W0, a_src0, a_dst0, b0, g0, be0,
         W1, a_src1, a_dst1, b1, g1, be1):
    xp = jnp.zeros((NP, D), jnp.float32).at[:N].set(x)
    src = edge_index[0].astype(jnp.int32)
    dst = edge_index[1].astype(jnp.int32)
    z2 = jnp.zeros((NP, HD), jnp.float32)
    z1 = jnp.zeros((NP,), jnp.float32)

    for (W, a_s, a_d, b, g, be) in (
            (W0, a_src0, a_dst0, b0, g0, be0),
            (W1, a_src1, a_dst1, b1, g1, be1)):
        va = a_s.reshape(D, 1)
        vd = a_d.reshape(D, 1)
        h0, h1, asrc, adst = _mm(xp, W, va, vd)
        hcat = jnp.concatenate([h0, h1], axis=0)
        out0, out1, s = _edge(src, dst, asrc.reshape(NP), adst.reshape(NP),
                              hcat, z2, z1)
        xp = _ln(xp, out0, out1, h0, h1, s.reshape(NP, 1), asrc, adst,
                 b.reshape(1, D), g.reshape(1, D), be.reshape(1, D))
    return xp[:N]


def kernel(x, edge_index, W0, a_src0, a_dst0, b0, g0, be0,
           W1, a_src1, a_dst1, b1, g1, be1):
    return _run(x, edge_index, W0, a_src0, a_dst0, b0, g0, be0,
                W1, a_src1, a_dst1, b1, g1, be1)
